# Initial kernel scaffold; baseline (speedup 1.0000x reference)
import jax
import jax.numpy as jnp
from jax.experimental import pallas as pl


def kernel(x, edge_index, W1, b1, W2, b2):
    raise NotImplementedError("write your pallas kernel here")


# speedup vs baseline: 9.3407x; 9.3407x over previous
import functools

import jax
import jax.numpy as jnp
from jax import lax
from jax.experimental import pallas as pl
from jax.experimental.pallas import tpu as pltpu
from jax.experimental.pallas import tpu_sc as plsc

F32 = jnp.float32
I32 = jnp.int32

NC = 2
NS = 16
NW = NC * NS
CH = 128
NR = 10240
RPT = NR // NS


def _mesh():
    return plsc.VectorSubcoreMesh(
        core_axis_name="c", subcore_axis_name="s", num_cores=NC, num_subcores=NS
    )


def _zero_rows(buf, nrows, width):
    zv = jnp.zeros((16,), F32)

    def row(i, _):
        for q in range(width // 16):
            buf[i, pl.ds(q * 16, 16)] = zv
        return 0

    lax.fori_loop(0, nrows, row, 0)


def _make_agg(k_per_w, dw):
    mesh = _mesh()
    scratch = [
        pltpu.VMEM((k_per_w, CH), I32),
        pltpu.VMEM((k_per_w, CH), I32),
        pltpu.VMEM((CH, dw), F32),
        pltpu.VMEM_SHARED((NR, dw), F32),
        pltpu.SemaphoreType.DMA,
    ]

    @functools.partial(
        pl.kernel,
        out_type=jax.ShapeDtypeStruct((NC * NR, dw), F32),
        mesh=mesh,
        scratch_types=scratch,
    )
    def agg(u, srcr, dstr, out, idx_s, idx_d, rows, acc, sem):
        c = lax.axis_index("c")
        t = lax.axis_index("s")
        w = c * NS + t
        gsrc = u
        _zero_rows(rows, CH, dw)
        for p in range(RPT // CH):
            pltpu.sync_copy(rows, acc.at[pl.ds(t * RPT + p * CH, CH)])
        plsc.subcore_barrier()
        pltpu.sync_copy(srcr.at[pl.ds(w * k_per_w, k_per_w)], idx_s)
        pltpu.sync_copy(dstr.at[pl.ds(w * k_per_w, k_per_w)], idx_d)

        def step(j, _):
            pltpu.async_copy(gsrc.at[idx_s.at[j]], rows, sem).wait()
            pltpu.sync_copy(rows, acc.at[idx_d.at[j]], add=True)
            return 0

        lax.fori_loop(0, k_per_w, step, 0)
        plsc.subcore_barrier()
        for p in range(RPT // CH):
            r0 = t * RPT + p * CH
            pltpu.sync_copy(acc.at[pl.ds(r0, CH)], rows)
            pltpu.sync_copy(rows, out.at[pl.ds(c * NR + r0, CH)])

    return agg


def _make_deg(k_per_w):
    mesh = _mesh()

    @functools.partial(
        pl.kernel,
        out_type=jax.ShapeDtypeStruct((NC * NR,), F32),
        mesh=mesh,
        scratch_types=[
            pltpu.VMEM((k_per_w, CH), I32),
            pltpu.VMEM((CH,), F32),
            pltpu.VMEM_SHARED((NR,), F32),
        ],
    )
    def deg(dstr, out, idx_d, ones, acc):
        c = lax.axis_index("c")
        t = lax.axis_index("s")
        w = c * NS + t
        zv = jnp.zeros((16,), F32)
        for q in range(CH // 16):
            ones[pl.ds(q * 16, 16)] = zv
        for p in range(RPT // CH):
            pltpu.sync_copy(ones, acc.at[pl.ds(t * RPT + p * CH, CH)])
        ov = jnp.ones((16,), F32)
        for q in range(CH // 16):
            ones[pl.ds(q * 16, 16)] = ov
        plsc.subcore_barrier()
        pltpu.sync_copy(dstr.at[pl.ds(w * k_per_w, k_per_w)], idx_d)

        def step(j, _):
            pltpu.sync_copy(ones, acc.at[idx_d.at[j]], add=True)
            return 0

        lax.fori_loop(0, k_per_w, step, 0)
        plsc.subcore_barrier()
        for p in range(RPT // CH):
            r0 = t * RPT + p * CH
            pltpu.sync_copy(acc.at[pl.ds(r0, CH)], ones)
            pltpu.sync_copy(ones, out.at[pl.ds(c * NR + r0, CH)])

    return deg


def _mm_body(x_ref, w_ref, o_ref):
    o_ref[...] = jnp.dot(x_ref[...], w_ref[...], preferred_element_type=F32)


def _scale_body(h_ref, da_ref, db_ref, u_ref, s_ref):
    s = lax.rsqrt(da_ref[...] + db_ref[...] + 1.0)
    s_ref[...] = s
    u_ref[...] = h_ref[...] * s


def _mid_body(aa_ref, ab_ref, u1_ref, s_ref, b1_ref, o_ref):
    h1 = (aa_ref[...] + ab_ref[...] + u1_ref[...]) * s_ref[...] + b1_ref[...]
    h1 = jnp.where(h1 >= 0, h1, 0.2 * h1)
    o_ref[...] = h1 * s_ref[...]


def _fin_body(qa_ref, qb_ref, v_ref, s_ref, w2_ref, b2_ref, o_ref):
    t = (qa_ref[...] + qb_ref[...] + v_ref[...]) * s_ref[...]
    o = jnp.dot(t, w2_ref[...], preferred_element_type=F32)
    o_ref[...] = o[:, 0:3] + b2_ref[...]


def kernel(x, edge_index, W1, b1, W2, b2):
    n, d_in = x.shape
    d_hid = W1.shape[1]
    d2 = 16
    e = edge_index.shape[1]
    k_per_w = -(-(-(-e // (NW * CH))) // 8) * 8
    ep = NW * CH * k_per_w

    src = edge_index[0].astype(I32)
    dst = edge_index[1].astype(I32)
    src_rows = jnp.concatenate([src, jnp.zeros((ep - e,), I32)]).reshape(-1, CH)
    dst_rows = jnp.concatenate([dst, jnp.full((ep - e,), n, I32)]).reshape(-1, CH)

    w2p = jnp.pad(W2, ((0, 0), (0, d2 - W2.shape[1])))
    b1r = b1.reshape(1, d_hid)
    b2r = b2.reshape(1, 3)

    degf = _make_deg(k_per_w)(dst_rows)

    h = pl.pallas_call(
        _mm_body, out_shape=jax.ShapeDtypeStruct((n, d_hid), F32)
    )(x, W1)

    u1, s = pl.pallas_call(
        _scale_body,
        out_shape=(
            jax.ShapeDtypeStruct((n, d_hid), F32),
            jax.ShapeDtypeStruct((n, 1), F32),
        ),
    )(h, degf[:n].reshape(n, 1), degf[NR : NR + n].reshape(n, 1))

    agg1 = _make_agg(k_per_w, d_hid)(u1, src_rows, dst_rows)

    v = pl.pallas_call(
        _mid_body, out_shape=jax.ShapeDtypeStruct((n, d_hid), F32)
    )(agg1[:n], agg1[NR : NR + n], u1, s, b1r)

    q = _make_agg(k_per_w, d_hid)(v, src_rows, dst_rows)

    out = pl.pallas_call(
        _fin_body, out_shape=jax.ShapeDtypeStruct((n, 3), F32)
    )(q[:n], q[NR : NR + n], v, s, w2p, b2r)
    return out


# speedup vs baseline: 9.5579x; 1.0232x over previous
import functools

import jax
import jax.numpy as jnp
from jax import lax
from jax.experimental import pallas as pl
from jax.experimental.pallas import tpu as pltpu
from jax.experimental.pallas import tpu_sc as plsc

F32 = jnp.float32
I32 = jnp.int32

NC = 2
NS = 16
NW = NC * NS
CH = 128
NR = 10240
RPT = NR // NS


def _mesh():
    return plsc.VectorSubcoreMesh(
        core_axis_name="c", subcore_axis_name="s", num_cores=NC, num_subcores=NS
    )


def _zero_rows(buf, nrows, width):
    zv = jnp.zeros((16,), F32)

    def row(i, _):
        for q in range(width // 16):
            buf[i, pl.ds(q * 16, 16)] = zv
        return 0

    lax.fori_loop(0, nrows, row, 0)


def _make_agg(k_per_w, dw):
    mesh = _mesh()
    nbuf = 2
    nphase = 2
    kidx = k_per_w // nphase
    ngroups = kidx // nbuf
    assert ngroups * nbuf * nphase == k_per_w
    scratch = [
        pltpu.VMEM((kidx, CH), I32),
        pltpu.VMEM((kidx, CH), I32),
    ]
    scratch += [pltpu.VMEM((CH, dw), F32) for _ in range(nbuf)]
    scratch.append(pltpu.VMEM_SHARED((NR, dw), F32))
    scratch += [pltpu.SemaphoreType.DMA for _ in range(nbuf)]

    @functools.partial(
        pl.kernel,
        out_type=jax.ShapeDtypeStruct((NC * NR, dw), F32),
        mesh=mesh,
        scratch_types=scratch,
    )
    def agg(u, srcr, dstr, out, *rest):
        idx_s, idx_d = rest[0], rest[1]
        rows = rest[2 : 2 + nbuf]
        acc = rest[2 + nbuf]
        sems = rest[3 + nbuf : 3 + 2 * nbuf]
        c = lax.axis_index("c")
        t = lax.axis_index("s")
        w = c * NS + t
        _zero_rows(rows[0], CH, dw)
        for p in range(RPT // CH):
            pltpu.sync_copy(rows[0], acc.at[pl.ds(t * RPT + p * CH, CH)])
        plsc.subcore_barrier()

        for ph in range(nphase):
            base = w * k_per_w + ph * kidx
            pltpu.sync_copy(srcr.at[pl.ds(base, kidx)], idx_s)
            pltpu.sync_copy(dstr.at[pl.ds(base, kidx)], idx_d)

            for b in range(nbuf):
                pltpu.async_copy(u.at[idx_s.at[b]], rows[b], sems[b])

            def group(g, _):
                for b in range(nbuf):
                    e = g * nbuf + b
                    pltpu.make_async_copy(
                        u.at[pl.ds(0, CH)], rows[b], sems[b]
                    ).wait()
                    pltpu.sync_copy(rows[b], acc.at[idx_d.at[e]], add=True)

                    @pl.when(g + 1 < ngroups)
                    def _():
                        pltpu.async_copy(u.at[idx_s.at[e + nbuf]], rows[b], sems[b])

                return 0

            lax.fori_loop(0, ngroups, group, 0)

        plsc.subcore_barrier()
        for p in range(RPT // CH):
            r0 = t * RPT + p * CH
            pltpu.sync_copy(acc.at[pl.ds(r0, CH)], rows[0])
            pltpu.sync_copy(rows[0], out.at[pl.ds(c * NR + r0, CH)])

    return agg


def _make_deg(k_per_w):
    mesh = _mesh()

    @functools.partial(
        pl.kernel,
        out_type=jax.ShapeDtypeStruct((NC * NR,), F32),
        mesh=mesh,
        scratch_types=[
            pltpu.VMEM((k_per_w, CH), I32),
            pltpu.VMEM((CH,), F32),
            pltpu.VMEM_SHARED((NR,), F32),
        ],
    )
    def deg(dstr, out, idx_d, ones, acc):
        c = lax.axis_index("c")
        t = lax.axis_index("s")
        w = c * NS + t
        zv = jnp.zeros((16,), F32)
        for q in range(CH // 16):
            ones[pl.ds(q * 16, 16)] = zv
        for p in range(RPT // CH):
            pltpu.sync_copy(ones, acc.at[pl.ds(t * RPT + p * CH, CH)])
        ov = jnp.ones((16,), F32)
        for q in range(CH // 16):
            ones[pl.ds(q * 16, 16)] = ov
        plsc.subcore_barrier()
        pltpu.sync_copy(dstr.at[pl.ds(w * k_per_w, k_per_w)], idx_d)

        def step(j, _):
            pltpu.sync_copy(ones, acc.at[idx_d.at[j]], add=True)
            return 0

        lax.fori_loop(0, k_per_w, step, 0)
        plsc.subcore_barrier()
        for p in range(RPT // CH):
            r0 = t * RPT + p * CH
            pltpu.sync_copy(acc.at[pl.ds(r0, CH)], ones)
            pltpu.sync_copy(ones, out.at[pl.ds(c * NR + r0, CH)])

    return deg


def _mm_scale_body(x_ref, w_ref, da_ref, db_ref, u_ref, s_ref):
    h = jnp.dot(x_ref[...], w_ref[...], preferred_element_type=F32)
    s = lax.rsqrt(da_ref[...] + db_ref[...] + 1.0)
    s_ref[...] = s
    u_ref[...] = h * s


def _mid_body(aa_ref, ab_ref, u1_ref, s_ref, b1_ref, o_ref):
    h1 = (aa_ref[...] + ab_ref[...] + u1_ref[...]) * s_ref[...] + b1_ref[...]
    h1 = jnp.where(h1 >= 0, h1, 0.2 * h1)
    o_ref[...] = h1 * s_ref[...]


def _fin_body(qa_ref, qb_ref, v_ref, s_ref, w2_ref, b2_ref, o_ref):
    t = (qa_ref[...] + qb_ref[...] + v_ref[...]) * s_ref[...]
    o = jnp.dot(t, w2_ref[...], preferred_element_type=F32)
    o_ref[...] = o[:, 0:3] + b2_ref[...]


def kernel(x, edge_index, W1, b1, W2, b2):
    n, d_in = x.shape
    d_hid = W1.shape[1]
    d2 = 16
    e = edge_index.shape[1]
    k_per_w = -(-(-(-e // (NW * CH))) // 8) * 8
    ep = NW * CH * k_per_w

    src = edge_index[0].astype(I32)
    dst = edge_index[1].astype(I32)
    src_rows = jnp.concatenate([src, jnp.zeros((ep - e,), I32)]).reshape(-1, CH)
    dst_rows = jnp.concatenate([dst, jnp.full((ep - e,), n, I32)]).reshape(-1, CH)

    w2p = jnp.pad(W2, ((0, 0), (0, d2 - W2.shape[1])))
    b1r = b1.reshape(1, d_hid)
    b2r = b2.reshape(1, 3)

    degf = _make_deg(k_per_w)(dst_rows)

    u1, s = pl.pallas_call(
        _mm_scale_body,
        out_shape=(
            jax.ShapeDtypeStruct((n, d_hid), F32),
            jax.ShapeDtypeStruct((n, 1), F32),
        ),
    )(x, W1, degf[:n].reshape(n, 1), degf[NR : NR + n].reshape(n, 1))

    agg1 = _make_agg(k_per_w, d_hid)(u1, src_rows, dst_rows)

    v = pl.pallas_call(
        _mid_body, out_shape=jax.ShapeDtypeStruct((n, d_hid), F32)
    )(agg1[:n], agg1[NR : NR + n], u1, s, b1r)

    q = _make_agg(k_per_w, d_hid)(v, src_rows, dst_rows)

    out = pl.pallas_call(
        _fin_body, out_shape=jax.ShapeDtypeStruct((n, 3), F32)
    )(q[:n], q[NR : NR + n], v, s, w2p, b2r)
    return out


# speedup vs baseline: 10.6465x; 1.1139x over previous
import functools

import jax
import jax.numpy as jnp
from jax import lax
from jax.experimental import pallas as pl
from jax.experimental.pallas import tpu as pltpu
from jax.experimental.pallas import tpu_sc as plsc

F32 = jnp.float32
I32 = jnp.int32

NC = 2
NS = 16
NW = NC * NS
CH = 128
NR = 10240
RPT = NR // NS


def _mesh():
    return plsc.VectorSubcoreMesh(
        core_axis_name="c", subcore_axis_name="s", num_cores=NC, num_subcores=NS
    )


def _zero_rows(buf, nrows, width):
    zv = jnp.zeros((16,), F32)

    def row(i, _):
        for q in range(width // 16):
            buf[i, pl.ds(q * 16, 16)] = zv
        return 0

    lax.fori_loop(0, nrows, row, 0)


def _make_agg(k_per_w, dw):
    mesh = _mesh()
    nbuf = 2
    nphase = 2
    kidx = k_per_w // nphase
    ngroups = kidx // nbuf
    assert ngroups * nbuf * nphase == k_per_w
    scratch = [
        pltpu.VMEM((kidx, CH), I32),
        pltpu.VMEM((kidx, CH), I32),
    ]
    scratch += [pltpu.VMEM((CH, dw), F32) for _ in range(nbuf)]
    scratch.append(pltpu.VMEM_SHARED((NR, dw), F32))
    scratch += [pltpu.SemaphoreType.DMA for _ in range(nbuf)]

    @functools.partial(
        pl.kernel,
        out_type=jax.ShapeDtypeStruct((NC * NR, dw), F32),
        mesh=mesh,
        scratch_types=scratch,
    )
    def agg(u, srcr, dstr, out, *rest):
        idx_s, idx_d = rest[0], rest[1]
        rows = rest[2 : 2 + nbuf]
        acc = rest[2 + nbuf]
        sems = rest[3 + nbuf : 3 + 2 * nbuf]
        c = lax.axis_index("c")
        t = lax.axis_index("s")
        w = c * NS + t
        _zero_rows(rows[0], CH, dw)
        for p in range(RPT // CH):
            pltpu.sync_copy(rows[0], acc.at[pl.ds(t * RPT + p * CH, CH)])
        plsc.subcore_barrier()

        for ph in range(nphase):
            base = w * k_per_w + ph * kidx
            pltpu.sync_copy(srcr.at[pl.ds(base, kidx)], idx_s)
            pltpu.sync_copy(dstr.at[pl.ds(base, kidx)], idx_d)

            for b in range(nbuf):
                pltpu.async_copy(u.at[idx_s.at[b]], rows[b], sems[b])

            def group(g, _):
                for b in range(nbuf):
                    e = g * nbuf + b
                    pltpu.make_async_copy(
                        u.at[pl.ds(0, CH)], rows[b], sems[b]
                    ).wait()
                    pltpu.sync_copy(rows[b], acc.at[idx_d.at[e]], add=True)

                    @pl.when(g + 1 < ngroups)
                    def _():
                        pltpu.async_copy(u.at[idx_s.at[e + nbuf]], rows[b], sems[b])

                return 0

            lax.fori_loop(0, ngroups, group, 0)

        plsc.subcore_barrier()
        for p in range(RPT // CH):
            r0 = t * RPT + p * CH
            pltpu.sync_copy(acc.at[pl.ds(r0, CH)], rows[0])
            pltpu.sync_copy(rows[0], out.at[pl.ds(c * NR + r0, CH)])

    return agg


def _make_deg(k_per_w):
    mesh = _mesh()

    @functools.partial(
        pl.kernel,
        out_type=jax.ShapeDtypeStruct((NC * NR,), F32),
        mesh=mesh,
        scratch_types=[
            pltpu.VMEM((k_per_w, CH), I32),
            pltpu.VMEM((CH,), F32),
            pltpu.VMEM_SHARED((NR,), F32),
        ],
    )
    def deg(dstr, out, idx_d, ones, acc):
        c = lax.axis_index("c")
        t = lax.axis_index("s")
        w = c * NS + t
        zv = jnp.zeros((16,), F32)
        for q in range(CH // 16):
            ones[pl.ds(q * 16, 16)] = zv
        for p in range(RPT // CH):
            pltpu.sync_copy(ones, acc.at[pl.ds(t * RPT + p * CH, CH)])
        ov = jnp.ones((16,), F32)
        for q in range(CH // 16):
            ones[pl.ds(q * 16, 16)] = ov
        plsc.subcore_barrier()
        pltpu.sync_copy(dstr.at[pl.ds(w * k_per_w, k_per_w)], idx_d)

        def step(j, _):
            pltpu.sync_copy(ones, acc.at[idx_d.at[j]], add=True)
            return 0

        lax.fori_loop(0, k_per_w, step, 0)
        plsc.subcore_barrier()
        for p in range(RPT // CH):
            r0 = t * RPT + p * CH
            pltpu.sync_copy(acc.at[pl.ds(r0, CH)], ones)
            pltpu.sync_copy(ones, out.at[pl.ds(c * NR + r0, CH)])

    return deg


def _mm_body(x_ref, w_ref, o_ref):
    o_ref[...] = jnp.dot(x_ref[...], w_ref[...], preferred_element_type=F32)


def _scale_body(h_ref, da_ref, db_ref, u_ref, s_ref):
    s = lax.rsqrt(da_ref[...] + db_ref[...] + 1.0)
    s_ref[...] = s
    u_ref[...] = h_ref[...] * s


def _mid_body(aa_ref, ab_ref, u1_ref, s_ref, b1_ref, o_ref):
    h1 = (aa_ref[...] + ab_ref[...] + u1_ref[...]) * s_ref[...] + b1_ref[...]
    h1 = jnp.where(h1 >= 0, h1, 0.2 * h1)
    o_ref[...] = h1 * s_ref[...]


def _fin_body(qa_ref, qb_ref, v_ref, s_ref, w2_ref, b2_ref, o_ref):
    t = (qa_ref[...] + qb_ref[...] + v_ref[...]) * s_ref[...]
    o = jnp.dot(t, w2_ref[...], preferred_element_type=F32)
    o_ref[...] = o[:, 0:3] + b2_ref[...]


def kernel(x, edge_index, W1, b1, W2, b2):
    n, d_in = x.shape
    d_hid = W1.shape[1]
    d2 = 16
    e = edge_index.shape[1]
    k_per_w = -(-(-(-e // (NW * CH))) // 8) * 8
    ep = NW * CH * k_per_w

    src = edge_index[0].astype(I32)
    dst = edge_index[1].astype(I32)
    src_rows = jnp.concatenate([src, jnp.zeros((ep - e,), I32)]).reshape(-1, CH)
    dst_rows = jnp.concatenate([dst, jnp.full((ep - e,), n, I32)]).reshape(-1, CH)

    w2p = jnp.pad(W2, ((0, 0), (0, d2 - W2.shape[1])))
    b1r = b1.reshape(1, d_hid)
    b2r = b2.reshape(1, 3)

    degf = _make_deg(k_per_w)(dst_rows)

    h = pl.pallas_call(
        _mm_body, out_shape=jax.ShapeDtypeStruct((n, d_hid), F32)
    )(x, W1)

    u1, s = pl.pallas_call(
        _scale_body,
        out_shape=(
            jax.ShapeDtypeStruct((n, d_hid), F32),
            jax.ShapeDtypeStruct((n, 1), F32),
        ),
    )(h, degf[:n].reshape(n, 1), degf[NR : NR + n].reshape(n, 1))

    agg1 = _make_agg(k_per_w, d_hid)(u1, src_rows, dst_rows)

    v = pl.pallas_call(
        _mid_body, out_shape=jax.ShapeDtypeStruct((n, d_hid), F32)
    )(agg1[:n], agg1[NR : NR + n], u1, s, b1r)

    q = _make_agg(k_per_w, d_hid)(v, src_rows, dst_rows)

    out = pl.pallas_call(
        _fin_body, out_shape=jax.ShapeDtypeStruct((n, 3), F32)
    )(q[:n], q[NR : NR + n], v, s, w2p, b2r)
    return out
